# Initial kernel scaffold; baseline (speedup 1.0000x reference)
import jax
import jax.numpy as jnp
from jax.experimental import pallas as pl


def kernel(output, target):
    raise NotImplementedError("write your pallas kernel here")


# speedup vs baseline: 38.0970x; 38.0970x over previous
import jax
import jax.numpy as jnp
from jax.experimental import pallas as pl
from jax.experimental.pallas import tpu as pltpu

_TIME = 4096
_BATCH = 1024
_NOUT = 2
_GAP = 8
_LANES = _BATCH * _NOUT
_TBLK = 256
_NSTEPS = _TIME // _TBLK


def _loss_kernel(o_ref, t_ref, out_ref, acc0_ref, acc1_ref):
    i = pl.program_id(0)

    o = o_ref[...]
    t = t_ref[...]

    d = o - t
    sq = d * d
    part0 = jnp.sum(sq.reshape(_TBLK // 8, 8, _LANES), axis=0)

    o3 = o.reshape(_TBLK // _GAP, _GAP, _LANES)
    rowsum = jnp.sum(o3, axis=1)
    tobs = t.reshape(_TBLK // _GAP, _GAP, _LANES)[:, 0, :]
    d1 = rowsum - 8.0 * tobs
    sq1 = d1 * d1
    part1 = jnp.sum(sq1.reshape(_TBLK // _GAP // 8, 8, _LANES), axis=0)

    @pl.when(i == 0)
    def _init():
        acc0_ref[...] = part0
        acc1_ref[...] = part1

    @pl.when(i > 0)
    def _accum():
        acc0_ref[...] += part0
        acc1_ref[...] += part1

    @pl.when(i == _NSTEPS - 1)
    def _finish():
        lane = jax.lax.broadcasted_iota(jnp.int32, (8, _LANES), 1)
        even = (lane % 2) == 0
        s0 = jnp.sum(jnp.where(even, acc0_ref[...], 0.0))
        s1 = jnp.sum(jnp.where(even, 0.0, acc1_ref[...]))
        n0 = float(_TIME * _BATCH)
        n1 = float((_TIME // _GAP) * _BATCH)
        out_ref[0, 0] = 0.5 * (s0 / n0) + 0.5 * (s1 / (64.0 * n1))


def kernel(output, target):
    o2 = output.reshape(_TIME, _LANES)
    t2 = target.reshape(_TIME, _LANES)
    out = pl.pallas_call(
        _loss_kernel,
        grid=(_NSTEPS,),
        in_specs=[
            pl.BlockSpec((_TBLK, _LANES), lambda i: (i, 0)),
            pl.BlockSpec((_TBLK, _LANES), lambda i: (i, 0)),
        ],
        out_specs=pl.BlockSpec(memory_space=pltpu.SMEM),
        out_shape=jax.ShapeDtypeStruct((1, 1), jnp.float32),
        scratch_shapes=[
            pltpu.VMEM((8, _LANES), jnp.float32),
            pltpu.VMEM((8, _LANES), jnp.float32),
        ],
    )(o2, t2)
    return out[0, 0]


# speedup vs baseline: 347.2598x; 9.1151x over previous
import jax
import jax.numpy as jnp
from jax.experimental import pallas as pl
from jax.experimental.pallas import tpu as pltpu

_TIME = 4096
_BATCH = 1024
_NOUT = 2
_GAP = 8
_ROWS = _TIME * 16
_TBLK = 256
_RBLK = _TBLK * 16
_NSTEPS = _TIME // _TBLK


def _loss_kernel(o_ref, t_ref, out_ref, acc0_ref, acc1_ref):
    i = pl.program_id(0)

    o = o_ref[...]
    t = t_ref[...]

    d = o - t
    sq = d * d
    part0 = jnp.sum(sq.reshape(_RBLK // 8, 8, 128), axis=0)

    o4 = o.reshape(_RBLK // 128, 8, 16, 128)
    rowsum = jnp.sum(o4, axis=1)
    tobs = t.reshape(_RBLK // 128, 8, 16, 128)[:, 0, :, :]
    d1 = rowsum - 8.0 * tobs
    sq1 = d1 * d1
    part1 = jnp.sum(sq1, axis=0)

    @pl.when(i == 0)
    def _init():
        acc0_ref[...] = part0
        acc1_ref[...] = part1

    @pl.when(i > 0)
    def _accum():
        acc0_ref[...] += part0
        acc1_ref[...] += part1

    @pl.when(i == _NSTEPS - 1)
    def _finish():
        row0 = jax.lax.broadcasted_iota(jnp.int32, (8, 128), 0)
        s0 = jnp.sum(jnp.where(row0 % 2 == 0, acc0_ref[...], 0.0))
        row1 = jax.lax.broadcasted_iota(jnp.int32, (16, 128), 0)
        s1 = jnp.sum(jnp.where(row1 % 2 == 1, acc1_ref[...], 0.0))
        n0 = float(_TIME * _BATCH)
        n1 = float((_TIME // _GAP) * _BATCH)
        out_ref[0, 0] = 0.5 * (s0 / n0) + 0.5 * (s1 / (64.0 * n1))


def _rowview(x):
    return (
        x.reshape(_TIME, 8, 128, _NOUT)
        .transpose(0, 1, 3, 2)
        .reshape(_ROWS, 128)
    )


def kernel(output, target):
    o2 = _rowview(output)
    t2 = _rowview(target)
    out = pl.pallas_call(
        _loss_kernel,
        grid=(_NSTEPS,),
        in_specs=[
            pl.BlockSpec((_RBLK, 128), lambda i: (i, 0)),
            pl.BlockSpec((_RBLK, 128), lambda i: (i, 0)),
        ],
        out_specs=pl.BlockSpec(memory_space=pltpu.SMEM),
        out_shape=jax.ShapeDtypeStruct((1, 1), jnp.float32),
        scratch_shapes=[
            pltpu.VMEM((8, 128), jnp.float32),
            pltpu.VMEM((16, 128), jnp.float32),
        ],
    )(o2, t2)
    return out[0, 0]


# speedup vs baseline: 389.4647x; 1.1215x over previous
import jax
import jax.numpy as jnp
from jax.experimental import pallas as pl
from jax.experimental.pallas import tpu as pltpu

_TIME = 4096
_BATCH = 1024
_NOUT = 2
_GAP = 8
_ROWS = _TIME * 16
_TBLK = 512
_RBLK = _TBLK * 16
_NSTEPS = _TIME // _TBLK


def _loss_kernel(o_ref, t_ref, out_ref, acc0_ref, acc1_ref):
    i = pl.program_id(0)

    o = o_ref[...]
    t = t_ref[...]

    d = o - t
    sq = d * d
    part0 = jnp.sum(sq.reshape(_RBLK // 8, 8, 128), axis=0)

    o4 = o.reshape(_RBLK // 128, 8, 16, 128)
    rowsum = jnp.sum(o4, axis=1)
    tobs = t.reshape(_RBLK // 128, 8, 16, 128)[:, 0, :, :]
    d1 = rowsum - 8.0 * tobs
    sq1 = d1 * d1
    part1 = jnp.sum(sq1, axis=0)

    @pl.when(i == 0)
    def _init():
        acc0_ref[...] = part0
        acc1_ref[...] = part1

    @pl.when(i > 0)
    def _accum():
        acc0_ref[...] += part0
        acc1_ref[...] += part1

    @pl.when(i == _NSTEPS - 1)
    def _finish():
        row0 = jax.lax.broadcasted_iota(jnp.int32, (8, 128), 0)
        s0 = jnp.sum(jnp.where(row0 % 2 == 0, acc0_ref[...], 0.0))
        row1 = jax.lax.broadcasted_iota(jnp.int32, (16, 128), 0)
        s1 = jnp.sum(jnp.where(row1 % 2 == 1, acc1_ref[...], 0.0))
        n0 = float(_TIME * _BATCH)
        n1 = float((_TIME // _GAP) * _BATCH)
        out_ref[0, 0] = 0.5 * (s0 / n0) + 0.5 * (s1 / (64.0 * n1))


def _rowview(x):
    return (
        x.reshape(_TIME, 8, 128, _NOUT)
        .transpose(0, 1, 3, 2)
        .reshape(_ROWS, 128)
    )


def kernel(output, target):
    o2 = _rowview(output)
    t2 = _rowview(target)
    out = pl.pallas_call(
        _loss_kernel,
        grid=(_NSTEPS,),
        in_specs=[
            pl.BlockSpec((_RBLK, 128), lambda i: (i, 0)),
            pl.BlockSpec((_RBLK, 128), lambda i: (i, 0)),
        ],
        out_specs=pl.BlockSpec(memory_space=pltpu.SMEM),
        out_shape=jax.ShapeDtypeStruct((1, 1), jnp.float32),
        scratch_shapes=[
            pltpu.VMEM((8, 128), jnp.float32),
            pltpu.VMEM((16, 128), jnp.float32),
        ],
    )(o2, t2)
    return out[0, 0]
